# baseline (speedup 1.0000x reference)
import functools

import jax
import jax.numpy as jnp
from jax import lax
from jax.experimental import pallas as pl
from jax.experimental.pallas import tpu as pltpu
from jax.experimental.pallas import tpu_sc as plsc

_NC = 2
_NS = 16
_NW = _NC * _NS


def _broadcast_table(w, b):
    t, d = w.shape
    rows = t // _NW
    half = t // _NC
    bs = b - b // 2

    mesh = plsc.VectorSubcoreMesh(core_axis_name="c", subcore_axis_name="s")

    @functools.partial(
        pl.kernel,
        mesh=mesh,
        out_type=jax.ShapeDtypeStruct((b, t, d), jnp.float32),
        scratch_types=[
            pltpu.VMEM((rows, d), jnp.float32),
            pltpu.VMEM_SHARED((half, d), jnp.float32),
            pltpu.SemaphoreType.DMA,
            pltpu.SemaphoreType.DMA,
        ] + [pltpu.SemaphoreType.DMA] * b,
    )
    def k(w_hbm, out_hbm, buf, stage, sem_r, sem_s, *sems):
        c = lax.axis_index("c")
        s = lax.axis_index("s")
        wid = s * _NC + c
        base = wid * rows

        rcp = pltpu.async_copy(w_hbm.at[pl.ds(base, rows)], buf, sem_r)
        stage_cp = pltpu.make_async_copy(
            w_hbm.at[pl.ds(c * half, half)], stage, sem_s
        )

        @pl.when(s == 0)
        def _start_stage():
            stage_cp.start()

        rcp.wait()
        copies = [
            pltpu.async_copy(buf, out_hbm.at[i, pl.ds(base, rows)], sems[i])
            for i in range(bs)
        ]

        @pl.when(s == 0)
        def _wait_stage():
            stage_cp.wait()

        plsc.subcore_barrier()

        sbase = c * half + s * rows
        spm = stage.at[pl.ds(s * rows, rows)]
        copies += [
            pltpu.async_copy(spm, out_hbm.at[i, pl.ds(sbase, rows)], sems[i])
            for i in range(bs, b)
        ]
        for cp in copies:
            cp.wait()

    return k(w)


def kernel(x, embed_weight):
    b = x.shape[0]
    return _broadcast_table(embed_weight, b)


# speedup vs baseline: 1.0958x; 1.0958x over previous
import functools

import jax
import jax.numpy as jnp
from jax import lax
from jax.experimental import pallas as pl
from jax.experimental.pallas import tpu as pltpu
from jax.experimental.pallas import tpu_sc as plsc

_NC = 2
_NS = 16
_NW = _NC * _NS
_NCHUNK = 4


def _broadcast_table(w, b):
    t, d = w.shape
    rows = t // _NW
    crows = rows // _NCHUNK

    mesh = plsc.VectorSubcoreMesh(core_axis_name="c", subcore_axis_name="s")

    @functools.partial(
        pl.kernel,
        mesh=mesh,
        out_type=jax.ShapeDtypeStruct((b, t, d), jnp.float32),
        scratch_types=[
            pltpu.VMEM((rows, d), jnp.float32),
        ] + [pltpu.SemaphoreType.DMA] * (_NCHUNK + b),
    )
    def k(w_hbm, out_hbm, buf, *sems):
        rsems, wsems = sems[:_NCHUNK], sems[_NCHUNK:]
        wid = lax.axis_index("s") * _NC + lax.axis_index("c")
        base = wid * rows

        reads = [
            pltpu.async_copy(
                w_hbm.at[pl.ds(base + j * crows, crows)],
                buf.at[pl.ds(j * crows, crows)],
                rsems[j],
            )
            for j in range(_NCHUNK)
        ]
        writes = []
        for j in range(_NCHUNK):
            reads[j].wait()
            writes += [
                pltpu.async_copy(
                    buf.at[pl.ds(j * crows, crows)],
                    out_hbm.at[i, pl.ds(base + j * crows, crows)],
                    wsems[i],
                )
                for i in range(b)
            ]
        for cp in writes:
            cp.wait()

    return k(w)


def kernel(x, embed_weight):
    b = x.shape[0]
    return _broadcast_table(embed_weight, b)


# speedup vs baseline: 1.1171x; 1.0194x over previous
import functools

import jax
import jax.numpy as jnp
from jax import lax
from jax.experimental import pallas as pl
from jax.experimental.pallas import tpu as pltpu
from jax.experimental.pallas import tpu_sc as plsc

_NC = 2
_NS = 16
_NW = _NC * _NS


def _broadcast_table(w, b):
    t, d = w.shape
    rows = t // _NW

    mesh = plsc.VectorSubcoreMesh(core_axis_name="c", subcore_axis_name="s")

    @functools.partial(
        pl.kernel,
        mesh=mesh,
        out_type=jax.ShapeDtypeStruct((b, t, d), jnp.float32),
        scratch_types=[
            pltpu.VMEM((rows, d), jnp.float32),
        ] + [pltpu.SemaphoreType.DMA] * b,
    )
    def k(w_hbm, out_hbm, buf, *sems):
        wid = lax.axis_index("s") * _NC + lax.axis_index("c")
        base = wid * rows
        pltpu.sync_copy(w_hbm.at[pl.ds(base, rows)], buf)
        copies = [
            pltpu.async_copy(buf, out_hbm.at[i, pl.ds(base, rows)], sems[i])
            for i in range(b)
        ]
        for c in copies:
            c.wait()

    return k(w)


def kernel(x, embed_weight):
    b = x.shape[0]
    return _broadcast_table(embed_weight, b)
